# Initial kernel scaffold; baseline (speedup 1.0000x reference)
import jax
import jax.numpy as jnp
from jax.experimental import pallas as pl


def kernel(x_seq, edge_index, edge_attr, enc_W1, enc_b1, enc_W2, enc_b2, enc_fcW, enc_fcb, lstm0_Wih, lstm0_Whh, lstm0_bih, lstm0_bhh, lstm1_Wih, lstm1_Whh, lstm1_bih, lstm1_bhh, head_W, head_b, dec_fcW, dec_fcb, dec_W1, dec_b1, dec_W2, dec_b2, dec_W3, dec_b3):
    raise NotImplementedError("write your pallas kernel here")


# speedup vs baseline: 1.0004x; 1.0004x over previous
import jax
import jax.numpy as jnp
from jax.experimental import pallas as pl


def _dec_fc_pallas(z, W, b):
    K, M = W.shape
    BK = 32000
    grid = (M // BK,)

    def body(z_ref, w_ref, b_ref, o_ref):
        o_ref[...] = jnp.maximum(
            jnp.dot(z_ref[...], w_ref[...], preferred_element_type=jnp.float32)
            + b_ref[...], 0.0)

    out = pl.pallas_call(
        body,
        grid=grid,
        in_specs=[
            pl.BlockSpec((1, K), lambda i: (0, 0)),
            pl.BlockSpec((K, BK), lambda i: (0, i)),
            pl.BlockSpec((1, BK), lambda i: (0, i)),
        ],
        out_specs=pl.BlockSpec((1, BK), lambda i: (0, i)),
        out_shape=jax.ShapeDtypeStruct((1, M), jnp.float32),
    )(z.reshape(1, K), W, b.reshape(1, M))
    return out.reshape(M)


def _self_loops(edge_index, ew, n):
    loop = jnp.arange(n, dtype=edge_index.dtype)
    src = jnp.concatenate([edge_index[0], loop])
    dst = jnp.concatenate([edge_index[1], loop])
    ew = jnp.concatenate([ew, jnp.ones((n,), ew.dtype)])
    return src, dst, ew


def _gcn(x, W, b, src, dst, ew, n):
    h = x @ W
    deg = jax.ops.segment_sum(ew, dst, num_segments=n)
    dinv = jnp.where(deg > 0, 1.0 / jnp.sqrt(deg), 0.0)
    coef = dinv[src] * ew * dinv[dst]
    agg = jax.ops.segment_sum(h[src] * coef[:, None], dst, num_segments=n)
    return agg + b


def _lstm_layer(xs, Wih, Whh, bih, bhh):
    Hh = Whh.shape[1]

    def step(carry, xt):
        h, c = carry
        g = xt @ Wih.T + h @ Whh.T + bih + bhh
        i, f, gg, o = jnp.split(g, 4)
        i = jax.nn.sigmoid(i)
        f = jax.nn.sigmoid(f)
        gg = jnp.tanh(gg)
        o = jax.nn.sigmoid(o)
        c = f * c + i * gg
        h = o * jnp.tanh(c)
        return (h, c), h

    init = (jnp.zeros((Hh,), xs.dtype), jnp.zeros((Hh,), xs.dtype))
    _, hs = jax.lax.scan(step, init, xs)
    return hs


def kernel(x_seq, edge_index, edge_attr, enc_W1, enc_b1, enc_W2, enc_b2,
           enc_fcW, enc_fcb, lstm0_Wih, lstm0_Whh, lstm0_bih, lstm0_bhh,
           lstm1_Wih, lstm1_Whh, lstm1_bih, lstm1_bhh, head_W, head_b,
           dec_fcW, dec_fcb, dec_W1, dec_b1, dec_W2, dec_b2, dec_W3, dec_b3):
    n = x_seq.shape[1]
    L = dec_W1.shape[0]
    src_w, dst_w, ew_w = _self_loops(edge_index, edge_attr, n)
    src_u, dst_u, ew_u = _self_loops(
        edge_index, jnp.ones((edge_index.shape[1],), jnp.float32), n)

    def encode(x):
        h = jax.nn.relu(_gcn(x, enc_W1, enc_b1, src_w, dst_w, ew_w, n))
        h = jax.nn.relu(_gcn(h, enc_W2, enc_b2, src_w, dst_w, ew_w, n))
        pooled = jnp.mean(h, axis=0)
        return pooled @ enc_fcW + enc_fcb

    lat = jax.vmap(encode)(x_seq)
    hs = _lstm_layer(lat, lstm0_Wih, lstm0_Whh, lstm0_bih, lstm0_bhh)
    hs = _lstm_layer(hs, lstm1_Wih, lstm1_Whh, lstm1_bih, lstm1_bhh)
    z = hs @ head_W + head_b
    agg = z[-1]
    x = _dec_fc_pallas(agg, dec_fcW, dec_fcb).reshape(n, L)
    h = jax.nn.relu(_gcn(x, dec_W1, dec_b1, src_u, dst_u, ew_u, n))
    h = jax.nn.relu(_gcn(h, dec_W2, dec_b2, src_u, dst_u, ew_u, n))
    out = _gcn(h, dec_W3, dec_b3, src_u, dst_u, ew_u, n)
    return out


# speedup vs baseline: 7.1148x; 7.1122x over previous
import functools

import jax
import jax.numpy as jnp
from jax import lax
from jax.experimental import pallas as pl
from jax.experimental.pallas import tpu as pltpu
from jax.experimental.pallas import tpu_sc as plsc

N = 10000
E = 320000
TT = 8
FD = 128
NW = 32
WPT = FD // NW
ROW = WPT * N

_SC_PARAMS = pltpu.CompilerParams(needs_layout_passes=False,
                                  use_tc_tiling_on_sc=False)


def _mesh():
    return plsc.VectorSubcoreMesh(core_axis_name="c", subcore_axis_name="s")


def _wid():
    return lax.axis_index("s") * 2 + lax.axis_index("c")


def _dg(a, b, dims):
    return lax.dot_general(a, b, (dims, ((), ())),
                           preferred_element_type=jnp.float32)


def _deg_body(sd_hbm, ew_hbm, out_hbm, sd_v, ew_v, dw_v, du_v):
    wid = _wid()
    epw = E // NW
    base = wid * epw
    pltpu.sync_copy(sd_hbm.at[pl.ds(base, epw)], sd_v)
    pltpu.sync_copy(ew_hbm.at[pl.ds(base, epw)], ew_v)

    def zero(i, _):
        dw_v[pl.ds(i * 16, 16)] = jnp.zeros((16,), jnp.float32)
        du_v[pl.ds(i * 16, 16)] = jnp.zeros((16,), jnp.float32)
        return 0

    lax.fori_loop(0, N // 16, zero, 0)
    ones = jnp.ones((16,), jnp.float32)

    def body(g, _):
        sd = sd_v[pl.ds(g * 16, 16)]
        dst = lax.shift_right_logical(sd, 16)
        w = ew_v[pl.ds(g * 16, 16)]
        plsc.addupdate_scatter(dw_v, [dst], w)
        plsc.addupdate_scatter(du_v, [dst], ones)
        return 0

    lax.fori_loop(0, epw // 16, body, 0)
    pltpu.sync_copy(dw_v, out_hbm.at[0, wid])
    pltpu.sync_copy(du_v, out_hbm.at[1, wid])


def _deg_partials(packed_sd, edge_attr):
    epw = E // NW
    return pl.kernel(
        _deg_body,
        out_type=jax.ShapeDtypeStruct((2, NW, N), jnp.float32),
        mesh=_mesh(),
        scratch_types=[
            pltpu.VMEM((epw,), jnp.int32),
            pltpu.VMEM((epw,), jnp.float32),
            pltpu.VMEM((N,), jnp.float32),
            pltpu.VMEM((N,), jnp.float32),
        ],
        compiler_params=_SC_PARAMS,
    )(packed_sd, edge_attr)


def _spmm_w_body(T, CH, table_hbm, sd_hbm, ew_hbm, out_hbm,
                 table_v, acc_v, sd_v, ew_v):
    wid = _wid()
    rbase = wid * ROW
    nch = E // CH
    gr = CH // 16

    def pass_t(t, _):
        pltpu.sync_copy(table_hbm.at[t, pl.ds(rbase, ROW)], table_v)
        pltpu.sync_copy(table_hbm.at[t, pl.ds(rbase, ROW)], acc_v)

        def chunk(c, _):
            pltpu.sync_copy(sd_hbm.at[pl.ds(c * CH, CH)], sd_v)
            pltpu.sync_copy(ew_hbm.at[pl.ds(c * CH, CH)], ew_v)

            def grp(g, _):
                sd = sd_v[pl.ds(g * 16, 16)]
                src = jnp.bitwise_and(sd, 0xFFFF)
                dst = lax.shift_right_logical(sd, 16)
                w = ew_v[pl.ds(g * 16, 16)]
                for f in range(WPT):
                    gv = plsc.load_gather(table_v, [src + f * N])
                    plsc.addupdate_scatter(acc_v, [dst + f * N], gv * w)
                return 0

            lax.fori_loop(0, gr, grp, 0)
            return 0

        lax.fori_loop(0, nch, chunk, 0)
        pltpu.sync_copy(acc_v, out_hbm.at[t, pl.ds(rbase, ROW)])
        return 0

    lax.fori_loop(0, T, pass_t, 0)


def _spmm_w(table, packed_sd, edge_attr):
    T = table.shape[0]
    CH = 20000
    body = functools.partial(_spmm_w_body, T, CH)
    return pl.kernel(
        body,
        out_type=jax.ShapeDtypeStruct((T, FD * N), jnp.float32),
        mesh=_mesh(),
        scratch_types=[
            pltpu.VMEM((ROW,), jnp.float32),
            pltpu.VMEM((ROW,), jnp.float32),
            pltpu.VMEM((CH,), jnp.int32),
            pltpu.VMEM((CH,), jnp.float32),
        ],
        compiler_params=_SC_PARAMS,
    )(table, packed_sd, edge_attr)


def _spmm_u_body(T, CH, table_hbm, sd_hbm, out_hbm, table_v, acc_v, sd_v):
    wid = _wid()
    rbase = wid * ROW
    nch = E // CH
    gr = CH // 16

    def pass_t(t, _):
        pltpu.sync_copy(table_hbm.at[t, pl.ds(rbase, ROW)], table_v)
        pltpu.sync_copy(table_hbm.at[t, pl.ds(rbase, ROW)], acc_v)

        def chunk(c, _):
            pltpu.sync_copy(sd_hbm.at[pl.ds(c * CH, CH)], sd_v)

            def grp(g, _):
                sd = sd_v[pl.ds(g * 16, 16)]
                src = jnp.bitwise_and(sd, 0xFFFF)
                dst = lax.shift_right_logical(sd, 16)
                for f in range(WPT):
                    gv = plsc.load_gather(table_v, [src + f * N])
                    plsc.addupdate_scatter(acc_v, [dst + f * N], gv)
                return 0

            lax.fori_loop(0, gr, grp, 0)
            return 0

        lax.fori_loop(0, nch, chunk, 0)
        pltpu.sync_copy(acc_v, out_hbm.at[t, pl.ds(rbase, ROW)])
        return 0

    lax.fori_loop(0, T, pass_t, 0)


def _spmm_u(table, packed_sd):
    T = table.shape[0]
    CH = 40000
    body = functools.partial(_spmm_u_body, T, CH)
    return pl.kernel(
        body,
        out_type=jax.ShapeDtypeStruct((T, FD * N), jnp.float32),
        mesh=_mesh(),
        scratch_types=[
            pltpu.VMEM((ROW,), jnp.float32),
            pltpu.VMEM((ROW,), jnp.float32),
            pltpu.VMEM((CH,), jnp.int32),
        ],
        compiler_params=_SC_PARAMS,
    )(table, packed_sd)


def _dinv_from_partials(parts):
    def body(p_ref, o_ref):
        s = jnp.sum(p_ref[...], axis=1) + 1.0
        o_ref[...] = lax.rsqrt(s)

    return pl.pallas_call(
        body,
        out_shape=jax.ShapeDtypeStruct((2, N), jnp.float32),
    )(parts)


def _enc_l1_tables(x_seq, W1, dinv_w):
    def body(x_ref, w_ref, d_ref, o_ref):
        h = _dg(w_ref[...], x_ref[0], ((0,), (1,)))
        o_ref[0] = h * d_ref[...]

    return pl.pallas_call(
        body,
        grid=(TT,),
        in_specs=[
            pl.BlockSpec((1, N, FD), lambda t: (t, 0, 0)),
            pl.BlockSpec((FD, FD), lambda t: (0, 0)),
            pl.BlockSpec((1, N), lambda t: (0, 0)),
        ],
        out_specs=pl.BlockSpec((1, FD, N), lambda t: (t, 0, 0)),
        out_shape=jax.ShapeDtypeStruct((TT, FD, N), jnp.float32),
    )(x_seq, W1, dinv_w)


def _mid_tables(acc, dinv, b, W):
    T = acc.shape[0]

    def body(a_ref, d_ref, b_ref, w_ref, o_ref):
        h = jnp.maximum(a_ref[0] * d_ref[...] + b_ref[...], 0.0)
        o_ref[0] = _dg(w_ref[...], h, ((0,), (0,))) * d_ref[...]

    return pl.pallas_call(
        body,
        grid=(T,),
        in_specs=[
            pl.BlockSpec((1, FD, N), lambda t: (t, 0, 0)),
            pl.BlockSpec((1, N), lambda t: (0, 0)),
            pl.BlockSpec((FD, 1), lambda t: (0, 0)),
            pl.BlockSpec((FD, FD), lambda t: (0, 0)),
        ],
        out_specs=pl.BlockSpec((1, FD, N), lambda t: (t, 0, 0)),
        out_shape=jax.ShapeDtypeStruct((T, FD, N), jnp.float32),
    )(acc, dinv, b, W)


def _enc_finish(acc, dinv_w, b2, fcW, fcb):
    L = fcW.shape[1]

    def body(a_ref, d_ref, b_ref, w_ref, c_ref, o_ref):
        h = jnp.maximum(a_ref[0] * d_ref[...] + b_ref[...], 0.0)
        pooled = jnp.sum(h, axis=1, keepdims=True) * (1.0 / N)
        o_ref[0] = _dg(w_ref[...], pooled, ((0,), (0,))) + c_ref[...]

    return pl.pallas_call(
        body,
        grid=(TT,),
        in_specs=[
            pl.BlockSpec((1, FD, N), lambda t: (t, 0, 0)),
            pl.BlockSpec((1, N), lambda t: (0, 0)),
            pl.BlockSpec((FD, 1), lambda t: (0, 0)),
            pl.BlockSpec((FD, L), lambda t: (0, 0)),
            pl.BlockSpec((L, 1), lambda t: (0, 0)),
        ],
        out_specs=pl.BlockSpec((1, L, 1), lambda t: (t, 0, 0)),
        out_shape=jax.ShapeDtypeStruct((TT, L, 1), jnp.float32),
    )(acc, dinv_w, b2, fcW, fcb)


def _lstm_head(lat, wih0, whh0, b0, wih1, whh1, b1, head_W, head_b):
    LH = whh0.shape[1]
    L = head_W.shape[1]

    def body(lat_ref, wi0, wh0, bb0, wi1, wh1, bb1, hw, hb, o_ref):
        def step(t, carry):
            h0, c0, h1, c1 = carry
            x = lat_ref[pl.ds(t, 1), :]
            g = (_dg(x, wi0[...], ((1,), (1,)))
                 + _dg(h0, wh0[...], ((1,), (1,))) + bb0[...])
            ii = jax.nn.sigmoid(g[:, 0:LH])
            ff = jax.nn.sigmoid(g[:, LH:2 * LH])
            gg = jnp.tanh(g[:, 2 * LH:3 * LH])
            oo = jax.nn.sigmoid(g[:, 3 * LH:4 * LH])
            c0 = ff * c0 + ii * gg
            h0 = oo * jnp.tanh(c0)
            g = (_dg(h0, wi1[...], ((1,), (1,)))
                 + _dg(h1, wh1[...], ((1,), (1,))) + bb1[...])
            ii = jax.nn.sigmoid(g[:, 0:LH])
            ff = jax.nn.sigmoid(g[:, LH:2 * LH])
            gg = jnp.tanh(g[:, 2 * LH:3 * LH])
            oo = jax.nn.sigmoid(g[:, 3 * LH:4 * LH])
            c1 = ff * c1 + ii * gg
            h1 = oo * jnp.tanh(c1)
            return (h0, c0, h1, c1)

        z = jnp.zeros((1, LH), jnp.float32)
        h0, c0, h1, c1 = lax.fori_loop(0, TT, step, (z, z, z, z))
        o_ref[...] = _dg(h1, hw[...], ((1,), (0,))) + hb[...]

    return pl.pallas_call(
        body,
        out_shape=jax.ShapeDtypeStruct((1, L), jnp.float32),
    )(lat, wih0, whh0, b0, wih1, whh1, b1, head_W, head_b)


def _dec_fc(z, W, b):
    K, M = W.shape
    BK = 32000

    def body(z_ref, w_ref, b_ref, o_ref):
        o_ref[...] = jnp.maximum(
            _dg(z_ref[...], w_ref[...], ((1,), (0,))) + b_ref[...], 0.0)

    out = pl.pallas_call(
        body,
        grid=(M // BK,),
        in_specs=[
            pl.BlockSpec((1, K), lambda i: (0, 0)),
            pl.BlockSpec((K, BK), lambda i: (0, i)),
            pl.BlockSpec((1, BK), lambda i: (0, i)),
        ],
        out_specs=pl.BlockSpec((1, BK), lambda i: (0, i)),
        out_shape=jax.ShapeDtypeStruct((1, M), jnp.float32),
    )(z, W, b.reshape(1, M))
    return out.reshape(M)


def _x_w1(X, W1):
    BN = 2000
    L = X.shape[1]

    def body(x_ref, w_ref, o_ref):
        o_ref[...] = _dg(x_ref[...], w_ref[...], ((1,), (0,)))

    return pl.pallas_call(
        body,
        grid=(N // BN,),
        in_specs=[
            pl.BlockSpec((BN, L), lambda i: (i, 0)),
            pl.BlockSpec((L, FD), lambda i: (0, 0)),
        ],
        out_specs=pl.BlockSpec((BN, FD), lambda i: (i, 0)),
        out_shape=jax.ShapeDtypeStruct((N, FD), jnp.float32),
    )(X, W1)


def _transpose_scale(X1, dinv_u):
    def body(x_ref, d_ref, o_ref):
        o_ref[...] = jnp.transpose(x_ref[...]) * d_ref[...]

    return pl.pallas_call(
        body,
        out_shape=jax.ShapeDtypeStruct((FD, N), jnp.float32),
    )(X1, dinv_u)


def _final(acc, dinv_u, b3):
    def body(a_ref, d_ref, b_ref, o_ref):
        o_ref[...] = jnp.transpose(a_ref[...] * d_ref[...] + b_ref[...])

    return pl.pallas_call(
        body,
        out_shape=jax.ShapeDtypeStruct((N, FD), jnp.float32),
    )(acc, dinv_u, b3)


def kernel(x_seq, edge_index, edge_attr, enc_W1, enc_b1, enc_W2, enc_b2,
           enc_fcW, enc_fcb, lstm0_Wih, lstm0_Whh, lstm0_bih, lstm0_bhh,
           lstm1_Wih, lstm1_Whh, lstm1_bih, lstm1_bhh, head_W, head_b,
           dec_fcW, dec_fcb, dec_W1, dec_b1, dec_W2, dec_b2, dec_W3, dec_b3):
    L = dec_W1.shape[0]
    packed_sd = jnp.bitwise_or(edge_index[0],
                               jnp.left_shift(edge_index[1], 16))

    parts = _deg_partials(packed_sd, edge_attr)
    dinv2 = _dinv_from_partials(parts)
    dinv_w = dinv2[0:1]
    dinv_u = dinv2[1:2]

    t1 = _enc_l1_tables(x_seq, enc_W1, dinv_w)
    a1 = _spmm_w(t1.reshape(TT, FD * N), packed_sd, edge_attr)
    t2 = _mid_tables(a1.reshape(TT, FD, N), dinv_w,
                     enc_b1.reshape(FD, 1), enc_W2)
    a2 = _spmm_w(t2.reshape(TT, FD * N), packed_sd, edge_attr)
    lat = _enc_finish(a2.reshape(TT, FD, N), dinv_w, enc_b2.reshape(FD, 1),
                      enc_fcW, enc_fcb.reshape(L, 1)).reshape(TT, L)

    aggz = _lstm_head(lat, lstm0_Wih, lstm0_Whh,
                      (lstm0_bih + lstm0_bhh).reshape(1, 4 * 128),
                      lstm1_Wih, lstm1_Whh,
                      (lstm1_bih + lstm1_bhh).reshape(1, 4 * 128),
                      head_W, head_b.reshape(1, L))

    xflat = _dec_fc(aggz, dec_fcW, dec_fcb)
    X = xflat.reshape(N, L)
    X1 = _x_w1(X, dec_W1)
    td1 = _transpose_scale(X1, dinv_u)
    ad1 = _spmm_u(td1.reshape(1, FD * N), packed_sd)
    td2 = _mid_tables(ad1.reshape(1, FD, N), dinv_u,
                      dec_b1.reshape(FD, 1), dec_W2)
    ad2 = _spmm_u(td2.reshape(1, FD * N), packed_sd)
    td3 = _mid_tables(ad2.reshape(1, FD, N), dinv_u,
                      dec_b2.reshape(FD, 1), dec_W3)
    ad3 = _spmm_u(td3.reshape(1, FD * N), packed_sd)
    out = _final(ad3.reshape(FD, N), dinv_u, dec_b3.reshape(FD, 1))
    return out


# speedup vs baseline: 18.8004x; 2.6424x over previous
import functools

import jax
import jax.numpy as jnp
from jax import lax
from jax.experimental import pallas as pl
from jax.experimental.pallas import tpu as pltpu
from jax.experimental.pallas import tpu_sc as plsc

N = 10000
E = 320000
TT = 8
FD = 128
NW = 32
WPT = FD // NW
ROW = WPT * N

_SC_PARAMS = pltpu.CompilerParams(needs_layout_passes=False,
                                  use_tc_tiling_on_sc=False)


def _mesh():
    return plsc.VectorSubcoreMesh(core_axis_name="c", subcore_axis_name="s")


def _wid():
    return lax.axis_index("s") * 2 + lax.axis_index("c")


def _dg(a, b, dims):
    return lax.dot_general(a, b, (dims, ((), ())),
                           preferred_element_type=jnp.float32)


def _deg_body(sd_hbm, ew_hbm, out_hbm, sd_v, ew_v, dw_v, du_v):
    wid = _wid()
    epw = E // NW
    base = wid * epw
    pltpu.sync_copy(sd_hbm.at[pl.ds(base, epw)], sd_v)
    pltpu.sync_copy(ew_hbm.at[pl.ds(base, epw)], ew_v)

    def zero(i, _):
        dw_v[pl.ds(i * 16, 16)] = jnp.zeros((16,), jnp.float32)
        du_v[pl.ds(i * 16, 16)] = jnp.zeros((16,), jnp.float32)
        return 0

    lax.fori_loop(0, N // 16, zero, 0)
    ones = jnp.ones((16,), jnp.float32)

    @plsc.parallel_loop(0, epw // 16, unroll=8)
    def _body(g):
        sd = sd_v[pl.ds(g * 16, 16)]
        dst = lax.shift_right_logical(sd, 16)
        w = ew_v[pl.ds(g * 16, 16)]
        plsc.addupdate_scatter(dw_v, [dst], w)
        plsc.addupdate_scatter(du_v, [dst], ones)
    pltpu.sync_copy(dw_v, out_hbm.at[0, wid])
    pltpu.sync_copy(du_v, out_hbm.at[1, wid])


def _deg_partials(packed_sd, edge_attr):
    epw = E // NW
    return pl.kernel(
        _deg_body,
        out_type=jax.ShapeDtypeStruct((2, NW, N), jnp.float32),
        mesh=_mesh(),
        scratch_types=[
            pltpu.VMEM((epw,), jnp.int32),
            pltpu.VMEM((epw,), jnp.float32),
            pltpu.VMEM((N,), jnp.float32),
            pltpu.VMEM((N,), jnp.float32),
        ],
        compiler_params=_SC_PARAMS,
    )(packed_sd, edge_attr)


def _spmm_w_body(T, CH, table_hbm, sd_hbm, ew_hbm, out_hbm,
                 table_v, acc_v, sd_v, ew_v):
    wid = _wid()
    rbase = wid * ROW
    nch = E // CH
    gr = CH // 16

    def pass_t(t, _):
        pltpu.sync_copy(table_hbm.at[t, pl.ds(rbase, ROW)], table_v)
        pltpu.sync_copy(table_hbm.at[t, pl.ds(rbase, ROW)], acc_v)

        def chunk(c, _):
            pltpu.sync_copy(sd_hbm.at[pl.ds(c * CH, CH)], sd_v)
            pltpu.sync_copy(ew_hbm.at[pl.ds(c * CH, CH)], ew_v)

            @plsc.parallel_loop(0, gr, unroll=8)
            def _grp(g):
                sd = sd_v[pl.ds(g * 16, 16)]
                src = jnp.bitwise_and(sd, 0xFFFF)
                dst = lax.shift_right_logical(sd, 16)
                w = ew_v[pl.ds(g * 16, 16)]
                for f in range(WPT):
                    gv = plsc.load_gather(table_v, [src + f * N])
                    plsc.addupdate_scatter(acc_v, [dst + f * N], gv * w)

            return 0

        lax.fori_loop(0, nch, chunk, 0)
        pltpu.sync_copy(acc_v, out_hbm.at[t, pl.ds(rbase, ROW)])
        return 0

    lax.fori_loop(0, T, pass_t, 0)


def _spmm_w(table, packed_sd, edge_attr):
    T = table.shape[0]
    CH = 20000
    body = functools.partial(_spmm_w_body, T, CH)
    return pl.kernel(
        body,
        out_type=jax.ShapeDtypeStruct((T, FD * N), jnp.float32),
        mesh=_mesh(),
        scratch_types=[
            pltpu.VMEM((ROW,), jnp.float32),
            pltpu.VMEM((ROW,), jnp.float32),
            pltpu.VMEM((CH,), jnp.int32),
            pltpu.VMEM((CH,), jnp.float32),
        ],
        compiler_params=_SC_PARAMS,
    )(table, packed_sd, edge_attr)


def _spmm_u_body(T, CH, table_hbm, sd_hbm, out_hbm, table_v, acc_v, sd_v):
    wid = _wid()
    rbase = wid * ROW
    nch = E // CH
    gr = CH // 16

    def pass_t(t, _):
        pltpu.sync_copy(table_hbm.at[t, pl.ds(rbase, ROW)], table_v)
        pltpu.sync_copy(table_hbm.at[t, pl.ds(rbase, ROW)], acc_v)

        def chunk(c, _):
            pltpu.sync_copy(sd_hbm.at[pl.ds(c * CH, CH)], sd_v)

            @plsc.parallel_loop(0, gr, unroll=8)
            def _grp(g):
                sd = sd_v[pl.ds(g * 16, 16)]
                src = jnp.bitwise_and(sd, 0xFFFF)
                dst = lax.shift_right_logical(sd, 16)
                for f in range(WPT):
                    gv = plsc.load_gather(table_v, [src + f * N])
                    plsc.addupdate_scatter(acc_v, [dst + f * N], gv)

            return 0

        lax.fori_loop(0, nch, chunk, 0)
        pltpu.sync_copy(acc_v, out_hbm.at[t, pl.ds(rbase, ROW)])
        return 0

    lax.fori_loop(0, T, pass_t, 0)


def _spmm_u(table, packed_sd):
    T = table.shape[0]
    CH = 40000
    body = functools.partial(_spmm_u_body, T, CH)
    return pl.kernel(
        body,
        out_type=jax.ShapeDtypeStruct((T, FD * N), jnp.float32),
        mesh=_mesh(),
        scratch_types=[
            pltpu.VMEM((ROW,), jnp.float32),
            pltpu.VMEM((ROW,), jnp.float32),
            pltpu.VMEM((CH,), jnp.int32),
        ],
        compiler_params=_SC_PARAMS,
    )(table, packed_sd)


def _dinv_from_partials(parts):
    def body(p_ref, o_ref):
        s = jnp.sum(p_ref[...], axis=1) + 1.0
        o_ref[...] = lax.rsqrt(s)

    return pl.pallas_call(
        body,
        out_shape=jax.ShapeDtypeStruct((2, N), jnp.float32),
    )(parts)


def _enc_l1_tables(x_seq, W1, dinv_w):
    def body(x_ref, w_ref, d_ref, o_ref):
        h = _dg(w_ref[...], x_ref[0], ((0,), (1,)))
        o_ref[0] = h * d_ref[...]

    return pl.pallas_call(
        body,
        grid=(TT,),
        in_specs=[
            pl.BlockSpec((1, N, FD), lambda t: (t, 0, 0)),
            pl.BlockSpec((FD, FD), lambda t: (0, 0)),
            pl.BlockSpec((1, N), lambda t: (0, 0)),
        ],
        out_specs=pl.BlockSpec((1, FD, N), lambda t: (t, 0, 0)),
        out_shape=jax.ShapeDtypeStruct((TT, FD, N), jnp.float32),
    )(x_seq, W1, dinv_w)


def _mid_tables(acc, dinv, b, W):
    T = acc.shape[0]

    def body(a_ref, d_ref, b_ref, w_ref, o_ref):
        h = jnp.maximum(a_ref[0] * d_ref[...] + b_ref[...], 0.0)
        o_ref[0] = _dg(w_ref[...], h, ((0,), (0,))) * d_ref[...]

    return pl.pallas_call(
        body,
        grid=(T,),
        in_specs=[
            pl.BlockSpec((1, FD, N), lambda t: (t, 0, 0)),
            pl.BlockSpec((1, N), lambda t: (0, 0)),
            pl.BlockSpec((FD, 1), lambda t: (0, 0)),
            pl.BlockSpec((FD, FD), lambda t: (0, 0)),
        ],
        out_specs=pl.BlockSpec((1, FD, N), lambda t: (t, 0, 0)),
        out_shape=jax.ShapeDtypeStruct((T, FD, N), jnp.float32),
    )(acc, dinv, b, W)


def _enc_finish(acc, dinv_w, b2, fcW, fcb):
    L = fcW.shape[1]

    def body(a_ref, d_ref, b_ref, w_ref, c_ref, o_ref):
        h = jnp.maximum(a_ref[0] * d_ref[...] + b_ref[...], 0.0)
        pooled = jnp.sum(h, axis=1, keepdims=True) * (1.0 / N)
        o_ref[0] = _dg(w_ref[...], pooled, ((0,), (0,))) + c_ref[...]

    return pl.pallas_call(
        body,
        grid=(TT,),
        in_specs=[
            pl.BlockSpec((1, FD, N), lambda t: (t, 0, 0)),
            pl.BlockSpec((1, N), lambda t: (0, 0)),
            pl.BlockSpec((FD, 1), lambda t: (0, 0)),
            pl.BlockSpec((FD, L), lambda t: (0, 0)),
            pl.BlockSpec((L, 1), lambda t: (0, 0)),
        ],
        out_specs=pl.BlockSpec((1, L, 1), lambda t: (t, 0, 0)),
        out_shape=jax.ShapeDtypeStruct((TT, L, 1), jnp.float32),
    )(acc, dinv_w, b2, fcW, fcb)


def _lstm_head(lat, wih0, whh0, b0, wih1, whh1, b1, head_W, head_b):
    LH = whh0.shape[1]
    L = head_W.shape[1]

    def body(lat_ref, wi0, wh0, bb0, wi1, wh1, bb1, hw, hb, o_ref):
        def step(t, carry):
            h0, c0, h1, c1 = carry
            x = lat_ref[pl.ds(t, 1), :]
            g = (_dg(x, wi0[...], ((1,), (1,)))
                 + _dg(h0, wh0[...], ((1,), (1,))) + bb0[...])
            ii = jax.nn.sigmoid(g[:, 0:LH])
            ff = jax.nn.sigmoid(g[:, LH:2 * LH])
            gg = jnp.tanh(g[:, 2 * LH:3 * LH])
            oo = jax.nn.sigmoid(g[:, 3 * LH:4 * LH])
            c0 = ff * c0 + ii * gg
            h0 = oo * jnp.tanh(c0)
            g = (_dg(h0, wi1[...], ((1,), (1,)))
                 + _dg(h1, wh1[...], ((1,), (1,))) + bb1[...])
            ii = jax.nn.sigmoid(g[:, 0:LH])
            ff = jax.nn.sigmoid(g[:, LH:2 * LH])
            gg = jnp.tanh(g[:, 2 * LH:3 * LH])
            oo = jax.nn.sigmoid(g[:, 3 * LH:4 * LH])
            c1 = ff * c1 + ii * gg
            h1 = oo * jnp.tanh(c1)
            return (h0, c0, h1, c1)

        z = jnp.zeros((1, LH), jnp.float32)
        h0, c0, h1, c1 = lax.fori_loop(0, TT, step, (z, z, z, z))
        o_ref[...] = _dg(h1, hw[...], ((1,), (0,))) + hb[...]

    return pl.pallas_call(
        body,
        out_shape=jax.ShapeDtypeStruct((1, L), jnp.float32),
    )(lat, wih0, whh0, b0, wih1, whh1, b1, head_W, head_b)


def _dec_fc(z, W, b):
    K, M = W.shape
    BK = 32000

    def body(z_ref, w_ref, b_ref, o_ref):
        o_ref[...] = jnp.maximum(
            _dg(z_ref[...], w_ref[...], ((1,), (0,))) + b_ref[...], 0.0)

    out = pl.pallas_call(
        body,
        grid=(M // BK,),
        in_specs=[
            pl.BlockSpec((1, K), lambda i: (0, 0)),
            pl.BlockSpec((K, BK), lambda i: (0, i)),
            pl.BlockSpec((1, BK), lambda i: (0, i)),
        ],
        out_specs=pl.BlockSpec((1, BK), lambda i: (0, i)),
        out_shape=jax.ShapeDtypeStruct((1, M), jnp.float32),
    )(z, W, b.reshape(1, M))
    return out.reshape(M)


def _x_w1(X, W1):
    BN = 2000
    L = X.shape[1]

    def body(x_ref, w_ref, o_ref):
        o_ref[...] = _dg(x_ref[...], w_ref[...], ((1,), (0,)))

    return pl.pallas_call(
        body,
        grid=(N // BN,),
        in_specs=[
            pl.BlockSpec((BN, L), lambda i: (i, 0)),
            pl.BlockSpec((L, FD), lambda i: (0, 0)),
        ],
        out_specs=pl.BlockSpec((BN, FD), lambda i: (i, 0)),
        out_shape=jax.ShapeDtypeStruct((N, FD), jnp.float32),
    )(X, W1)


def _transpose_scale(X1, dinv_u):
    def body(x_ref, d_ref, o_ref):
        o_ref[...] = jnp.transpose(x_ref[...]) * d_ref[...]

    return pl.pallas_call(
        body,
        out_shape=jax.ShapeDtypeStruct((FD, N), jnp.float32),
    )(X1, dinv_u)


def _final(acc, dinv_u, b3):
    def body(a_ref, d_ref, b_ref, o_ref):
        o_ref[...] = jnp.transpose(a_ref[...] * d_ref[...] + b_ref[...])

    return pl.pallas_call(
        body,
        out_shape=jax.ShapeDtypeStruct((N, FD), jnp.float32),
    )(acc, dinv_u, b3)


def kernel(x_seq, edge_index, edge_attr, enc_W1, enc_b1, enc_W2, enc_b2,
           enc_fcW, enc_fcb, lstm0_Wih, lstm0_Whh, lstm0_bih, lstm0_bhh,
           lstm1_Wih, lstm1_Whh, lstm1_bih, lstm1_bhh, head_W, head_b,
           dec_fcW, dec_fcb, dec_W1, dec_b1, dec_W2, dec_b2, dec_W3, dec_b3):
    L = dec_W1.shape[0]
    packed_sd = jnp.bitwise_or(edge_index[0],
                               jnp.left_shift(edge_index[1], 16))

    parts = _deg_partials(packed_sd, edge_attr)
    dinv2 = _dinv_from_partials(parts)
    dinv_w = dinv2[0:1]
    dinv_u = dinv2[1:2]

    t1 = _enc_l1_tables(x_seq, enc_W1, dinv_w)
    a1 = _spmm_w(t1.reshape(TT, FD * N), packed_sd, edge_attr)
    t2 = _mid_tables(a1.reshape(TT, FD, N), dinv_w,
                     enc_b1.reshape(FD, 1), enc_W2)
    a2 = _spmm_w(t2.reshape(TT, FD * N), packed_sd, edge_attr)
    lat = _enc_finish(a2.reshape(TT, FD, N), dinv_w, enc_b2.reshape(FD, 1),
                      enc_fcW, enc_fcb.reshape(L, 1)).reshape(TT, L)

    aggz = _lstm_head(lat, lstm0_Wih, lstm0_Whh,
                      (lstm0_bih + lstm0_bhh).reshape(1, 4 * 128),
                      lstm1_Wih, lstm1_Whh,
                      (lstm1_bih + lstm1_bhh).reshape(1, 4 * 128),
                      head_W, head_b.reshape(1, L))

    xflat = _dec_fc(aggz, dec_fcW, dec_fcb)
    X = xflat.reshape(N, L)
    X1 = _x_w1(X, dec_W1)
    td1 = _transpose_scale(X1, dinv_u)
    ad1 = _spmm_u(td1.reshape(1, FD * N), packed_sd)
    td2 = _mid_tables(ad1.reshape(1, FD, N), dinv_u,
                      dec_b1.reshape(FD, 1), dec_W2)
    ad2 = _spmm_u(td2.reshape(1, FD * N), packed_sd)
    td3 = _mid_tables(ad2.reshape(1, FD, N), dinv_u,
                      dec_b2.reshape(FD, 1), dec_W3)
    ad3 = _spmm_u(td3.reshape(1, FD * N), packed_sd)
    out = _final(ad3.reshape(FD, N), dinv_u, dec_b3.reshape(FD, 1))
    return out


# speedup vs baseline: 22.2436x; 1.1831x over previous
import functools

import jax
import jax.numpy as jnp
from jax import lax
from jax.experimental import pallas as pl
from jax.experimental.pallas import tpu as pltpu
from jax.experimental.pallas import tpu_sc as plsc

N = 10000
E = 320000
TT = 8
FD = 128
NW = 32
WPT = FD // NW
ROW = WPT * N

_SC_PARAMS = pltpu.CompilerParams(needs_layout_passes=False,
                                  use_tc_tiling_on_sc=False)


def _mesh():
    return plsc.VectorSubcoreMesh(core_axis_name="c", subcore_axis_name="s")


def _wid():
    return lax.axis_index("s") * 2 + lax.axis_index("c")


def _dg(a, b, dims):
    return lax.dot_general(a, b, (dims, ((), ())),
                           preferred_element_type=jnp.float32)


def _deg_body(sd_hbm, ew_hbm, out_hbm, sd_v, ew_v, dw_v, du_v):
    wid = _wid()
    epw = E // NW
    base = wid * epw
    pltpu.sync_copy(sd_hbm.at[pl.ds(base, epw)], sd_v)
    pltpu.sync_copy(ew_hbm.at[pl.ds(base, epw)], ew_v)

    def zero(i, _):
        dw_v[pl.ds(i * 16, 16)] = jnp.zeros((16,), jnp.float32)
        du_v[pl.ds(i * 16, 16)] = jnp.zeros((16,), jnp.float32)
        return 0

    lax.fori_loop(0, N // 16, zero, 0)
    ones = jnp.ones((16,), jnp.float32)

    @plsc.parallel_loop(0, epw // 16, unroll=8)
    def _body(g):
        sd = sd_v[pl.ds(g * 16, 16)]
        dst = lax.shift_right_logical(sd, 16)
        w = ew_v[pl.ds(g * 16, 16)]
        plsc.addupdate_scatter(dw_v, [dst], w)
        plsc.addupdate_scatter(du_v, [dst], ones)
    pltpu.sync_copy(dw_v, out_hbm.at[0, wid])
    pltpu.sync_copy(du_v, out_hbm.at[1, wid])


def _deg_partials(packed_sd, edge_attr):
    epw = E // NW
    return pl.kernel(
        _deg_body,
        out_type=jax.ShapeDtypeStruct((2, NW, N), jnp.float32),
        mesh=_mesh(),
        scratch_types=[
            pltpu.VMEM((epw,), jnp.int32),
            pltpu.VMEM((epw,), jnp.float32),
            pltpu.VMEM((N,), jnp.float32),
            pltpu.VMEM((N,), jnp.float32),
        ],
        compiler_params=_SC_PARAMS,
    )(packed_sd, edge_attr)


def _spmm_w_body(T, CH, table_hbm, sd_hbm, ew_hbm, out_hbm,
                 table_v, acc_v, sd_v, ew_v, sem0, sem1):
    wid = _wid()
    rbase = wid * ROW
    nch = E // CH
    gr = CH // 16
    sems = (sem0, sem1)

    def start(c, b):
        pltpu.make_async_copy(sd_hbm.at[pl.ds(c * CH, CH)], sd_v.at[b],
                              sems[b]).start()
        pltpu.make_async_copy(ew_hbm.at[pl.ds(c * CH, CH)], ew_v.at[b],
                              sems[b]).start()

    def wait(b):
        pltpu.make_async_copy(sd_hbm.at[pl.ds(0, CH)], sd_v.at[b],
                              sems[b]).wait()
        pltpu.make_async_copy(ew_hbm.at[pl.ds(0, CH)], ew_v.at[b],
                              sems[b]).wait()

    def process(b):
        @plsc.parallel_loop(0, gr, unroll=8)
        def _grp(g):
            sd = sd_v[b, pl.ds(g * 16, 16)]
            srcv = jnp.bitwise_and(sd, 0xFFFF)
            dstv = lax.shift_right_logical(sd, 16)
            w = ew_v[b, pl.ds(g * 16, 16)]
            for f in range(WPT):
                gv = plsc.load_gather(table_v, [srcv + f * N])
                plsc.addupdate_scatter(acc_v, [dstv + f * N], gv * w)

    start(0, 0)

    def pass_t(t, _):
        pltpu.sync_copy(table_hbm.at[t, pl.ds(rbase, ROW)], table_v)
        pltpu.sync_copy(table_hbm.at[t, pl.ds(rbase, ROW)], acc_v)

        def pair(i, _):
            c0 = 2 * i
            start(c0 + 1, 1)
            wait(0)
            process(0)
            start(lax.rem(c0 + 2, nch), 0)
            wait(1)
            process(1)
            return 0

        lax.fori_loop(0, nch // 2, pair, 0)
        pltpu.sync_copy(acc_v, out_hbm.at[t, pl.ds(rbase, ROW)])
        return 0

    lax.fori_loop(0, T, pass_t, 0)
    wait(0)


def _spmm_w(table, packed_sd, edge_attr):
    T = table.shape[0]
    CH = 10000
    body = functools.partial(_spmm_w_body, T, CH)
    return pl.kernel(
        body,
        out_type=jax.ShapeDtypeStruct((T, FD * N), jnp.float32),
        mesh=_mesh(),
        scratch_types=[
            pltpu.VMEM((ROW,), jnp.float32),
            pltpu.VMEM((ROW,), jnp.float32),
            pltpu.VMEM((2, CH), jnp.int32),
            pltpu.VMEM((2, CH), jnp.float32),
            pltpu.SemaphoreType.DMA,
            pltpu.SemaphoreType.DMA,
        ],
        compiler_params=_SC_PARAMS,
    )(table, packed_sd, edge_attr)


def _spmm_u_body(T, CH, table_hbm, sd_hbm, out_hbm,
                 table_v, acc_v, sd_v, sem0, sem1):
    wid = _wid()
    rbase = wid * ROW
    nch = E // CH
    gr = CH // 16
    sems = (sem0, sem1)

    def start(c, b):
        pltpu.make_async_copy(sd_hbm.at[pl.ds(c * CH, CH)], sd_v.at[b],
                              sems[b]).start()

    def wait(b):
        pltpu.make_async_copy(sd_hbm.at[pl.ds(0, CH)], sd_v.at[b],
                              sems[b]).wait()

    def process(b):
        @plsc.parallel_loop(0, gr, unroll=8)
        def _grp(g):
            sd = sd_v[b, pl.ds(g * 16, 16)]
            srcv = jnp.bitwise_and(sd, 0xFFFF)
            dstv = lax.shift_right_logical(sd, 16)
            for f in range(WPT):
                gv = plsc.load_gather(table_v, [srcv + f * N])
                plsc.addupdate_scatter(acc_v, [dstv + f * N], gv)

    start(0, 0)

    def pass_t(t, _):
        pltpu.sync_copy(table_hbm.at[t, pl.ds(rbase, ROW)], table_v)
        pltpu.sync_copy(table_hbm.at[t, pl.ds(rbase, ROW)], acc_v)

        def pair(i, _):
            c0 = 2 * i
            start(c0 + 1, 1)
            wait(0)
            process(0)
            start(lax.rem(c0 + 2, nch), 0)
            wait(1)
            process(1)
            return 0

        lax.fori_loop(0, nch // 2, pair, 0)
        pltpu.sync_copy(acc_v, out_hbm.at[t, pl.ds(rbase, ROW)])
        return 0

    lax.fori_loop(0, T, pass_t, 0)
    wait(0)


def _spmm_u(table, packed_sd):
    T = table.shape[0]
    CH = 20000
    body = functools.partial(_spmm_u_body, T, CH)
    return pl.kernel(
        body,
        out_type=jax.ShapeDtypeStruct((T, FD * N), jnp.float32),
        mesh=_mesh(),
        scratch_types=[
            pltpu.VMEM((ROW,), jnp.float32),
            pltpu.VMEM((ROW,), jnp.float32),
            pltpu.VMEM((2, CH), jnp.int32),
            pltpu.SemaphoreType.DMA,
            pltpu.SemaphoreType.DMA,
        ],
        compiler_params=_SC_PARAMS,
    )(table, packed_sd)


def _dinv_from_partials(parts):
    def body(p_ref, o_ref):
        s = jnp.sum(p_ref[...], axis=1) + 1.0
        o_ref[...] = lax.rsqrt(s)

    return pl.pallas_call(
        body,
        out_shape=jax.ShapeDtypeStruct((2, N), jnp.float32),
    )(parts)


def _enc_l1_tables(x_seq, W1, dinv_w):
    def body(x_ref, w_ref, d_ref, o_ref):
        h = _dg(w_ref[...], x_ref[0], ((0,), (1,)))
        o_ref[0] = h * d_ref[...]

    return pl.pallas_call(
        body,
        grid=(TT,),
        in_specs=[
            pl.BlockSpec((1, N, FD), lambda t: (t, 0, 0)),
            pl.BlockSpec((FD, FD), lambda t: (0, 0)),
            pl.BlockSpec((1, N), lambda t: (0, 0)),
        ],
        out_specs=pl.BlockSpec((1, FD, N), lambda t: (t, 0, 0)),
        out_shape=jax.ShapeDtypeStruct((TT, FD, N), jnp.float32),
    )(x_seq, W1, dinv_w)


def _mid_tables(acc, dinv, b, W):
    T = acc.shape[0]

    def body(a_ref, d_ref, b_ref, w_ref, o_ref):
        h = jnp.maximum(a_ref[0] * d_ref[...] + b_ref[...], 0.0)
        o_ref[0] = _dg(w_ref[...], h, ((0,), (0,))) * d_ref[...]

    return pl.pallas_call(
        body,
        grid=(T,),
        in_specs=[
            pl.BlockSpec((1, FD, N), lambda t: (t, 0, 0)),
            pl.BlockSpec((1, N), lambda t: (0, 0)),
            pl.BlockSpec((FD, 1), lambda t: (0, 0)),
            pl.BlockSpec((FD, FD), lambda t: (0, 0)),
        ],
        out_specs=pl.BlockSpec((1, FD, N), lambda t: (t, 0, 0)),
        out_shape=jax.ShapeDtypeStruct((T, FD, N), jnp.float32),
    )(acc, dinv, b, W)


def _enc_finish(acc, dinv_w, b2, fcW, fcb):
    L = fcW.shape[1]

    def body(a_ref, d_ref, b_ref, w_ref, c_ref, o_ref):
        h = jnp.maximum(a_ref[0] * d_ref[...] + b_ref[...], 0.0)
        pooled = jnp.sum(h, axis=1, keepdims=True) * (1.0 / N)
        o_ref[0] = _dg(w_ref[...], pooled, ((0,), (0,))) + c_ref[...]

    return pl.pallas_call(
        body,
        grid=(TT,),
        in_specs=[
            pl.BlockSpec((1, FD, N), lambda t: (t, 0, 0)),
            pl.BlockSpec((1, N), lambda t: (0, 0)),
            pl.BlockSpec((FD, 1), lambda t: (0, 0)),
            pl.BlockSpec((FD, L), lambda t: (0, 0)),
            pl.BlockSpec((L, 1), lambda t: (0, 0)),
        ],
        out_specs=pl.BlockSpec((1, L, 1), lambda t: (t, 0, 0)),
        out_shape=jax.ShapeDtypeStruct((TT, L, 1), jnp.float32),
    )(acc, dinv_w, b2, fcW, fcb)


def _lstm_head(lat, wih0, whh0, b0, wih1, whh1, b1, head_W, head_b):
    LH = whh0.shape[1]
    L = head_W.shape[1]

    def body(lat_ref, wi0, wh0, bb0, wi1, wh1, bb1, hw, hb, o_ref):
        def step(t, carry):
            h0, c0, h1, c1 = carry
            x = lat_ref[pl.ds(t, 1), :]
            g = (_dg(x, wi0[...], ((1,), (1,)))
                 + _dg(h0, wh0[...], ((1,), (1,))) + bb0[...])
            ii = jax.nn.sigmoid(g[:, 0:LH])
            ff = jax.nn.sigmoid(g[:, LH:2 * LH])
            gg = jnp.tanh(g[:, 2 * LH:3 * LH])
            oo = jax.nn.sigmoid(g[:, 3 * LH:4 * LH])
            c0 = ff * c0 + ii * gg
            h0 = oo * jnp.tanh(c0)
            g = (_dg(h0, wi1[...], ((1,), (1,)))
                 + _dg(h1, wh1[...], ((1,), (1,))) + bb1[...])
            ii = jax.nn.sigmoid(g[:, 0:LH])
            ff = jax.nn.sigmoid(g[:, LH:2 * LH])
            gg = jnp.tanh(g[:, 2 * LH:3 * LH])
            oo = jax.nn.sigmoid(g[:, 3 * LH:4 * LH])
            c1 = ff * c1 + ii * gg
            h1 = oo * jnp.tanh(c1)
            return (h0, c0, h1, c1)

        z = jnp.zeros((1, LH), jnp.float32)
        h0, c0, h1, c1 = lax.fori_loop(0, TT, step, (z, z, z, z))
        o_ref[...] = _dg(h1, hw[...], ((1,), (0,))) + hb[...]

    return pl.pallas_call(
        body,
        out_shape=jax.ShapeDtypeStruct((1, L), jnp.float32),
    )(lat, wih0, whh0, b0, wih1, whh1, b1, head_W, head_b)


def _dec_fc(z, W, b):
    K, M = W.shape
    BK = 32000

    def body(z_ref, w_ref, b_ref, o_ref):
        o_ref[...] = jnp.maximum(
            _dg(z_ref[...], w_ref[...], ((1,), (0,))) + b_ref[...], 0.0)

    out = pl.pallas_call(
        body,
        grid=(M // BK,),
        in_specs=[
            pl.BlockSpec((1, K), lambda i: (0, 0)),
            pl.BlockSpec((K, BK), lambda i: (0, i)),
            pl.BlockSpec((1, BK), lambda i: (0, i)),
        ],
        out_specs=pl.BlockSpec((1, BK), lambda i: (0, i)),
        out_shape=jax.ShapeDtypeStruct((1, M), jnp.float32),
    )(z, W, b.reshape(1, M))
    return out.reshape(M)


def _x_w1(X, W1):
    BN = 2000
    L = X.shape[1]

    def body(x_ref, w_ref, o_ref):
        o_ref[...] = _dg(x_ref[...], w_ref[...], ((1,), (0,)))

    return pl.pallas_call(
        body,
        grid=(N // BN,),
        in_specs=[
            pl.BlockSpec((BN, L), lambda i: (i, 0)),
            pl.BlockSpec((L, FD), lambda i: (0, 0)),
        ],
        out_specs=pl.BlockSpec((BN, FD), lambda i: (i, 0)),
        out_shape=jax.ShapeDtypeStruct((N, FD), jnp.float32),
    )(X, W1)


def _transpose_scale(X1, dinv_u):
    def body(x_ref, d_ref, o_ref):
        o_ref[...] = jnp.transpose(x_ref[...]) * d_ref[...]

    return pl.pallas_call(
        body,
        out_shape=jax.ShapeDtypeStruct((FD, N), jnp.float32),
    )(X1, dinv_u)


def _final(acc, dinv_u, b3):
    def body(a_ref, d_ref, b_ref, o_ref):
        o_ref[...] = jnp.transpose(a_ref[...] * d_ref[...] + b_ref[...])

    return pl.pallas_call(
        body,
        out_shape=jax.ShapeDtypeStruct((N, FD), jnp.float32),
    )(acc, dinv_u, b3)


def kernel(x_seq, edge_index, edge_attr, enc_W1, enc_b1, enc_W2, enc_b2,
           enc_fcW, enc_fcb, lstm0_Wih, lstm0_Whh, lstm0_bih, lstm0_bhh,
           lstm1_Wih, lstm1_Whh, lstm1_bih, lstm1_bhh, head_W, head_b,
           dec_fcW, dec_fcb, dec_W1, dec_b1, dec_W2, dec_b2, dec_W3, dec_b3):
    L = dec_W1.shape[0]
    packed_sd = jnp.bitwise_or(edge_index[0],
                               jnp.left_shift(edge_index[1], 16))

    parts = _deg_partials(packed_sd, edge_attr)
    dinv2 = _dinv_from_partials(parts)
    dinv_w = dinv2[0:1]
    dinv_u = dinv2[1:2]

    t1 = _enc_l1_tables(x_seq, enc_W1, dinv_w)
    a1 = _spmm_w(t1.reshape(TT, FD * N), packed_sd, edge_attr)
    t2 = _mid_tables(a1.reshape(TT, FD, N), dinv_w,
                     enc_b1.reshape(FD, 1), enc_W2)
    a2 = _spmm_w(t2.reshape(TT, FD * N), packed_sd, edge_attr)
    lat = _enc_finish(a2.reshape(TT, FD, N), dinv_w, enc_b2.reshape(FD, 1),
                      enc_fcW, enc_fcb.reshape(L, 1)).reshape(TT, L)

    aggz = _lstm_head(lat, lstm0_Wih, lstm0_Whh,
                      (lstm0_bih + lstm0_bhh).reshape(1, 4 * 128),
                      lstm1_Wih, lstm1_Whh,
                      (lstm1_bih + lstm1_bhh).reshape(1, 4 * 128),
                      head_W, head_b.reshape(1, L))

    xflat = _dec_fc(aggz, dec_fcW, dec_fcb)
    X = xflat.reshape(N, L)
    X1 = _x_w1(X, dec_W1)
    td1 = _transpose_scale(X1, dinv_u)
    ad1 = _spmm_u(td1.reshape(1, FD * N), packed_sd)
    td2 = _mid_tables(ad1.reshape(1, FD, N), dinv_u,
                      dec_b1.reshape(FD, 1), dec_W2)
    ad2 = _spmm_u(td2.reshape(1, FD * N), packed_sd)
    td3 = _mid_tables(ad2.reshape(1, FD, N), dinv_u,
                      dec_b2.reshape(FD, 1), dec_W3)
    ad3 = _spmm_u(td3.reshape(1, FD * N), packed_sd)
    out = _final(ad3.reshape(FD, N), dinv_u, dec_b3.reshape(FD, 1))
    return out
